# baseline (device time: 229384 ns/iter reference)
import jax
import jax.numpy as jnp
from jax import lax
from jax.experimental import pallas as pl
from jax.experimental.pallas import tpu as pltpu

N_DEV = 8
_GELU_C = 0.7978845608028654

_PART_MASKS = ((1, 3, 4), (3, 4, 1), (4, 1, 3))


def _gelu(y):
    return 0.5 * y * (1.0 + jnp.tanh(_GELU_C * (y + 0.044715 * y * y * y)))


def kernel(x, w_mat):
    m_per, k = x.shape
    _, n_per = w_mat.shape

    base = m_per // 3 // 8 * 8
    part_rows = (m_per - 2 * base, base, base)
    part_offs = (0, part_rows[0], part_rows[0] + base)

    def body(x_ref, w_ref, out_ref, buf_a, buf_b, buf_c,
             send_sems, recv_sems, credits):
        my = lax.axis_index("i")
        bufs = (buf_a, buf_b, buf_c)

        barrier_sem = pltpu.get_barrier_semaphore()
        for mask in (1, 3, 4):
            pl.semaphore_signal(
                barrier_sem, inc=1,
                device_id=(my ^ mask,), device_id_type=pl.DeviceIdType.MESH,
            )
        pl.semaphore_wait(barrier_sem, 3)

        def x_part(p):
            return x_ref.at[pl.ds(part_offs[p], part_rows[p]), :]

        def src_ref(p, b):
            return x_part(p) if b == 0 else bufs[p].at[b - 1]

        def gemm(p, slot_idx, origin, r0=0, nrows=None):
            nrows = part_rows[p] if nrows is None else nrows
            out_ref[pl.ds(origin * m_per + part_offs[p] + r0, nrows), :] = (
                _gelu(jnp.dot(bufs[p][slot_idx, pl.ds(r0, nrows)], w_ref[...],
                              preferred_element_type=jnp.float32))
            )

        def rdma(p, j, b, dst_idx, sem=None, rows=None):
            sem = (1 << j) - 1 + b if sem is None else sem
            src = src_ref(p, b)
            dst = bufs[p].at[dst_idx]
            if rows is not None:
                src = bufs[p].at[b - 1, pl.ds(*rows)]
                dst = bufs[p].at[dst_idx, pl.ds(*rows)]
            return pltpu.make_async_remote_copy(
                src_ref=src, dst_ref=dst,
                send_sem=send_sems.at[p, sem],
                recv_sem=recv_sems.at[p, sem],
                device_id=(my ^ _PART_MASKS[p][j],),
                device_id_type=pl.DeviceIdType.MESH,
            )

        masks = _PART_MASKS
        h1 = [88, 88, 88]
        h2 = [part_rows[p] - h1[p] for p in range(3)]
        s0 = [rdma(p, 0, 0, 0) for p in range(3)]
        s1 = [[rdma(p, 1, b, 1 + b) for b in range(2)] for p in range(3)]
        s2 = [[rdma(p, 2, b, 3 + b % 2) for b in range(3)] for p in range(3)]
        s2b3 = [(rdma(p, 2, 3, 4, sem=6, rows=(0, h1[p])),
                 rdma(p, 2, 3, 4, sem=7, rows=(h1[p], h2[p])))
                for p in range(3)]
        all_rdmas = s0 + [r for pp in s1 for r in pp] + \
            [r for pp in s2 for r in pp] + [r for pp in s2b3 for r in pp]

        for p in range(3):
            s0[p].start()
        for p in range(3):
            s1[p][0].start()

        out_ref[pl.ds(my * m_per, m_per), :] = _gelu(
            jnp.dot(x_ref[...], w_ref[...], preferred_element_type=jnp.float32)
        )

        for p in range(3):
            s0[p].wait_recv()
        for p in range(3):
            s1[p][1].start()
        for p in range(3):
            s2[p][0].start()
        for p in range(3):
            s2[p][1].start()
        for p in range(3):
            gemm(p, 0, my ^ masks[p][0])

        for p in range(3):
            s1[p][0].wait_recv()
        for p in range(3):
            gemm(p, 1, my ^ masks[p][1])
        for p in range(3):
            s1[p][1].wait_recv()
        for p in range(3):
            gemm(p, 2, my ^ masks[p][1] ^ masks[p][0])

        for b in range(3):
            for p in (1, 2, 0):
                m2 = masks[p][2]
                s2[p][b].wait_recv()
                org = (my ^ m2) ^ (0, masks[p][0], masks[p][1])[b]
                gemm(p, 3 + b % 2, org)
                if b < 2:
                    pl.semaphore_signal(
                        credits.at[p], inc=1,
                        device_id=(my ^ m2,),
                        device_id_type=pl.DeviceIdType.MESH,
                    )
            if b == 0:
                for p in (1, 2, 0):
                    pl.semaphore_wait(credits.at[p], 1)
                    s2[p][2].start()
            elif b == 1:
                for p in (1, 2, 0):
                    pl.semaphore_wait(credits.at[p], 1)
                    s2b3[p][0].start()
                    s2b3[p][1].start()

        for h in range(2):
            for p in (1, 2, 0):
                org = (my ^ masks[p][2]) ^ masks[p][0] ^ masks[p][1]
                s2b3[p][h].wait_recv()
                gemm(p, 4, org, 0 if h == 0 else h1[p],
                     h1[p] if h == 0 else h2[p])

        for r in all_rdmas:
            r.wait_send()

    return pl.pallas_call(
        body,
        out_shape=jax.ShapeDtypeStruct((N_DEV * m_per, n_per), jnp.float32),
        in_specs=[
            pl.BlockSpec(memory_space=pltpu.VMEM),
            pl.BlockSpec(memory_space=pltpu.VMEM),
        ],
        out_specs=pl.BlockSpec(memory_space=pltpu.VMEM),
        scratch_shapes=[
            pltpu.VMEM((5, part_rows[0], k), jnp.float32),
            pltpu.VMEM((5, part_rows[1], k), jnp.float32),
            pltpu.VMEM((5, part_rows[2], k), jnp.float32),
            pltpu.SemaphoreType.DMA((3, 8)),
            pltpu.SemaphoreType.DMA((3, 8)),
            pltpu.SemaphoreType.REGULAR((3,)),
        ],
        compiler_params=pltpu.CompilerParams(
            collective_id=0,
            vmem_limit_bytes=100 * 1024 * 1024,
        ),
    )(x, w_mat)
